# baseline (speedup 1.0000x reference)
import functools

import jax
import jax.numpy as jnp
from jax import lax
from jax.experimental import pallas as pl
from jax.experimental.pallas import tpu as pltpu
from jax.experimental.pallas import tpu_sc as plsc

_B = 4096
_V = 100000
_D = 64
_L = 16
_UNROLL = 4


@jax.jit
def _gmf(user_ids, item_ids, user_table, item_table):
    info = plsc.get_sparse_core_info()
    nc, ns = info.num_cores, info.num_subcores
    nw = nc * ns
    d_per_w = _D // nw

    utT = user_table.T
    itT = item_table.T

    mesh = plsc.VectorSubcoreMesh(core_axis_name="c", subcore_axis_name="s")

    @functools.partial(
        pl.kernel,
        mesh=mesh,
        out_type=jax.ShapeDtypeStruct((_D, _B), jnp.float32),
        scratch_types=[
            pltpu.VMEM((_B,), jnp.int32),
            pltpu.VMEM((_B,), jnp.int32),
            pltpu.VMEM((1, _V), jnp.float32),
            pltpu.VMEM((_B,), jnp.float32),
            pltpu.VMEM((d_per_w, _B), jnp.float32),
            pltpu.SemaphoreType.DMA,
            pltpu.SemaphoreType.DMA,
        ],
        compiler_params=pltpu.CompilerParams(needs_layout_passes=False),
    )
    def k(uid_hbm, iid_hbm, utT_hbm, itT_hbm, outT_hbm,
          uids_v, iids_v, col_v, ugath_v, orows_v, sem_ids, sem_out):
        wid = lax.axis_index("s") * nc + lax.axis_index("c")
        cu_ids = pltpu.async_copy(uid_hbm, uids_v, sem_ids)
        ci_ids = pltpu.async_copy(iid_hbm, iids_v, sem_ids)

        zeros = jnp.zeros((_L,), jnp.int32)
        out_copies = []

        for dd in range(d_per_w):
            d = wid * d_per_w + dd
            pltpu.sync_copy(utT_hbm.at[pl.ds(d, 1), :], col_v)
            if dd == 0:
                cu_ids.wait()
                ci_ids.wait()

            def gath_u(kk, carry):
                for q in range(_UNROLL):
                    s = pl.ds((kk * _UNROLL + q) * _L, _L)
                    ugath_v[s] = plsc.load_gather(col_v, [zeros, uids_v[s]])
                return carry

            lax.fori_loop(0, _B // (_L * _UNROLL), gath_u, 0)
            pltpu.sync_copy(itT_hbm.at[pl.ds(d, 1), :], col_v)

            def gath_i(kk, carry):
                for q in range(_UNROLL):
                    s = pl.ds((kk * _UNROLL + q) * _L, _L)
                    orows_v[dd, s] = ugath_v[s] * plsc.load_gather(
                        col_v, [zeros, iids_v[s]])
                return carry

            lax.fori_loop(0, _B // (_L * _UNROLL), gath_i, 0)
            out_copies.append(pltpu.async_copy(
                orows_v.at[pl.ds(dd, 1), :],
                outT_hbm.at[pl.ds(d, 1), :], sem_out))

        for c in out_copies:
            c.wait()

    outT = k(user_ids, item_ids, utT, itT)
    return outT.T


def kernel(user_ids, item_ids, user_table, item_table):
    return _gmf(user_ids, item_ids, user_table, item_table)
